# baseline (speedup 1.0000x reference)
import jax
import jax.numpy as jnp
from jax import lax
from jax.experimental import pallas as pl
from jax.experimental.pallas import tpu as pltpu
from jax.experimental.pallas import tpu_sc as plsc

VOCAB_SIZE = 100000
D_MODEL = 128
MAX_POS = 2048
BATCH = 4
SEQ_LEN = 2048

_NUM_WORKERS = 32
_SBLK = SEQ_LEN // _NUM_WORKERS
_LANES = 16
_NPAIR = BATCH // 2


def _emb_kernel(x_hbm, tok_hbm, pos_hbm, out_hbm, idx_v, tok_v, pos_v,
                sem_g, sem_w, sem_p, sem_i):
    wid = lax.axis_index("s") * 2 + lax.axis_index("c")
    s_base = wid * _SBLK

    idx_cps = [
        pltpu.async_copy(
            x_hbm.at[b, pl.ds(s_base, _SBLK)],
            idx_v.at[b // 2, pl.ds((b % 2) * _SBLK, _SBLK)],
            sem_i,
        )
        for b in range(BATCH)
    ]
    gathers = []
    for p in range(_NPAIR):
        idx_cps[2 * p].wait()
        idx_cps[2 * p + 1].wait()
        gathers.append(
            pltpu.async_copy(
                tok_hbm.at[idx_v.at[p]],
                tok_v.at[pl.ds(p * 2 * _SBLK, 2 * _SBLK)],
                sem_g.at[p],
            )
        )

    pltpu.async_copy(pos_hbm.at[pl.ds(s_base, _SBLK)], pos_v, sem_p).wait()

    writes = []
    for p in range(_NPAIR):
        gathers[p].wait()

        @pl.loop(0, _SBLK, unroll=2)
        def _add_row(r):
            t0 = p * 2 * _SBLK + r
            for j in range(D_MODEL // _LANES):
                sl = pl.ds(j * _LANES, _LANES)
                v = pos_v[r, sl]
                plsc.addupdate(tok_v.at[t0, sl], v)
                plsc.addupdate(tok_v.at[t0 + _SBLK, sl], v)

        for h in range(2):
            b = p * 2 + h
            writes.append(
                pltpu.async_copy(
                    tok_v.at[pl.ds(b * _SBLK, _SBLK)],
                    out_hbm.at[pl.ds(b * SEQ_LEN + s_base, _SBLK)],
                    sem_w.at[b],
                )
            )

    for w in writes:
        w.wait()


@jax.jit
def kernel(x, token_emb, pos_emb):
    mesh = plsc.VectorSubcoreMesh(core_axis_name="c", subcore_axis_name="s")
    run = pl.kernel(
        _emb_kernel,
        out_type=jax.ShapeDtypeStruct((BATCH * SEQ_LEN, D_MODEL), jnp.float32),
        mesh=mesh,
        scratch_types=[
            pltpu.VMEM((_NPAIR, 2 * _SBLK), jnp.int32),
            pltpu.VMEM((BATCH * _SBLK, D_MODEL), jnp.float32),
            pltpu.VMEM((_SBLK, D_MODEL), jnp.float32),
            pltpu.SemaphoreType.DMA((_NPAIR,)),
            pltpu.SemaphoreType.DMA((BATCH,)),
            pltpu.SemaphoreType.DMA,
            pltpu.SemaphoreType.DMA,
        ],
    )
    out = run(x, token_emb, pos_emb)
    return out.reshape(BATCH, SEQ_LEN, D_MODEL)


# speedup vs baseline: 1.0081x; 1.0081x over previous
import jax
import jax.numpy as jnp
from jax import lax
from jax.experimental import pallas as pl
from jax.experimental.pallas import tpu as pltpu
from jax.experimental.pallas import tpu_sc as plsc

VOCAB_SIZE = 100000
D_MODEL = 128
MAX_POS = 2048
BATCH = 4
SEQ_LEN = 2048

_NUM_WORKERS = 32
_SBLK = SEQ_LEN // _NUM_WORKERS
_LANES = 16
_NPAIR = BATCH // 2


def _emb_kernel(x_hbm, tok_hbm, pos_hbm, out_hbm, idx_v, tok_v, pos_v,
                sem_g, sem_w, sem_p, sem_i):
    wid = lax.axis_index("s") * 2 + lax.axis_index("c")
    s_base = wid * _SBLK

    idx_cps = [
        pltpu.async_copy(
            x_hbm.at[b, pl.ds(s_base, _SBLK)],
            idx_v.at[b // 2, pl.ds((b % 2) * _SBLK, _SBLK)],
            sem_i,
        )
        for b in range(BATCH)
    ]
    gathers = []
    for p in range(_NPAIR):
        idx_cps[2 * p].wait()
        idx_cps[2 * p + 1].wait()
        gathers.append(
            pltpu.async_copy(
                tok_hbm.at[idx_v.at[p]],
                tok_v.at[pl.ds(p * 2 * _SBLK, 2 * _SBLK)],
                sem_g.at[p],
            )
        )

    pltpu.async_copy(pos_hbm.at[pl.ds(s_base, _SBLK)], pos_v, sem_p).wait()

    writes = []
    for p in range(_NPAIR):
        gathers[p].wait()

        @pl.loop(0, _SBLK, unroll=1)
        def _add_row(r):
            t0 = p * 2 * _SBLK + r
            for j in range(D_MODEL // _LANES):
                sl = pl.ds(j * _LANES, _LANES)
                v = pos_v[r, sl]
                plsc.addupdate(tok_v.at[t0, sl], v)
                plsc.addupdate(tok_v.at[t0 + _SBLK, sl], v)

        for h in range(2):
            b = p * 2 + h
            writes.append(
                pltpu.async_copy(
                    tok_v.at[pl.ds(b * _SBLK, _SBLK)],
                    out_hbm.at[pl.ds(b * SEQ_LEN + s_base, _SBLK)],
                    sem_w.at[b],
                )
            )

    for w in writes:
        w.wait()


@jax.jit
def kernel(x, token_emb, pos_emb):
    mesh = plsc.VectorSubcoreMesh(core_axis_name="c", subcore_axis_name="s")
    run = pl.kernel(
        _emb_kernel,
        out_type=jax.ShapeDtypeStruct((BATCH * SEQ_LEN, D_MODEL), jnp.float32),
        mesh=mesh,
        scratch_types=[
            pltpu.VMEM((_NPAIR, 2 * _SBLK), jnp.int32),
            pltpu.VMEM((BATCH * _SBLK, D_MODEL), jnp.float32),
            pltpu.VMEM((_SBLK, D_MODEL), jnp.float32),
            pltpu.SemaphoreType.DMA((_NPAIR,)),
            pltpu.SemaphoreType.DMA((BATCH,)),
            pltpu.SemaphoreType.DMA,
            pltpu.SemaphoreType.DMA,
        ],
    )
    out = run(x, token_emb, pos_emb)
    return out.reshape(BATCH, SEQ_LEN, D_MODEL)
